# baseline (speedup 1.0000x reference)
import functools

import jax
import jax.numpy as jnp
from jax import lax
from jax.experimental import pallas as pl
from jax.experimental.pallas import tpu as pltpu
from jax.experimental.pallas import tpu_sc as plsc

N = 10000
E = 320000
N2 = 20000
E2 = 640000
A = 40000
G = 256
D_FEAT = 128
D = 64
NI2 = 16
NC = 10

NCORES = 2
NSUB = 16
NW = NCORES * NSUB

NPAD = 10240
N2PAD = 20480
EPAD = 327680
APAD = 49152
E2PAD = 655360

@functools.lru_cache(maxsize=None)
def _make_segsum(n_pad, e_pad, chunk):
    per_w = e_pad // NW
    n_chunks = per_w // chunk
    assert n_chunks % 2 == 0 and n_chunks >= 2
    krow = chunk // 128
    rows_per_tile = n_pad // NSUB
    mesh = plsc.VectorSubcoreMesh(core_axis_name="c", subcore_axis_name="s")

    @functools.partial(
        pl.kernel,
        out_type=jax.ShapeDtypeStruct((NCORES, n_pad, D), jnp.float32),
        mesh=mesh,
        scratch_types=[
            pltpu.VMEM((2 * krow, 128), jnp.int32),
            pltpu.VMEM((2 * krow, 128), jnp.int32),
            pltpu.VMEM((chunk, D), jnp.float32),
            pltpu.VMEM((chunk, D), jnp.float32),
            pltpu.VMEM_SHARED((n_pad, D), jnp.float32),
            pltpu.SemaphoreType.DMA,
            pltpu.SemaphoreType.DMA,
            [pltpu.SemaphoreType.DMA] * 4,
            [pltpu.SemaphoreType.DMA] * 4,
        ],
        compiler_params=pltpu.CompilerParams(use_tc_tiling_on_sc=False),
    )
    def segsum(y_hbm, eidx_hbm, zeros_hbm, out_hbm,
               idx0_v, idx1_v, rows0_v, rows1_v, acc_s,
               gsem0, gsem1, ssem0, ssem1):
        cid = lax.axis_index("c")
        sid = lax.axis_index("s")
        wid = cid * NSUB + sid
        tile_base = sid * rows_per_tile
        base_g = wid * n_chunks * 2 * krow

        def fire(row0, idx_v, rows_v, sem):
            pltpu.sync_copy(eidx_hbm.at[pl.ds(row0, 2 * krow)], idx_v)
            for j in range(krow):
                pltpu.async_copy(y_hbm.at[idx_v.at[j]],
                                 rows_v.at[pl.ds(j * 128, 128)], sem)

        def drain(rows_v, sem):
            pltpu.make_async_copy(y_hbm.at[pl.ds(0, chunk)], rows_v, sem).wait()

        def scatter(idx_v, rows_v, ssem):
            for j in range(krow):
                pltpu.async_copy(rows_v.at[pl.ds(j * 128, 128)],
                                 acc_s.at[idx_v.at[krow + j]], ssem[j],
                                 add=True)

        def drain_s(idx_v, rows_v, ssem):
            for j in range(krow):
                pltpu.make_async_copy(rows_v.at[pl.ds(j * 128, 128)],
                                      acc_s.at[idx_v.at[krow + j]],
                                      ssem[j]).wait()

        fire(base_g, idx0_v, rows0_v, gsem0)
        pltpu.sync_copy(zeros_hbm,
                        acc_s.at[pl.ds(tile_base, rows_per_tile)])
        plsc.subcore_barrier()

        @pl.loop(0, n_chunks // 2)
        def _(h):
            ci1 = 2 * h + 1
            ci2 = 2 * h + 2
            drain(rows0_v, gsem0)

            @pl.when(h > 0)
            def _():
                drain_s(idx1_v, rows1_v, ssem1)

            fire(base_g + ci1 * 2 * krow, idx1_v, rows1_v, gsem1)
            scatter(idx0_v, rows0_v, ssem0)
            drain(rows1_v, gsem1)

            @pl.when(ci2 < n_chunks)
            def _():
                drain_s(idx0_v, rows0_v, ssem0)
                fire(base_g + ci2 * 2 * krow, idx0_v, rows0_v, gsem0)

            scatter(idx1_v, rows1_v, ssem1)

        drain_s(idx0_v, rows0_v, ssem0)
        drain_s(idx1_v, rows1_v, ssem1)
        plsc.subcore_barrier()
        pltpu.sync_copy(
            acc_s.at[pl.ds(tile_base, rows_per_tile)],
            out_hbm.at[cid].at[pl.ds(tile_base, rows_per_tile)])

    return segsum


NHALF = 10240
NLOC = 10368


@functools.lru_cache(maxsize=None)
def _make_segsum_split(e_pad, chunk):
    per_w = e_pad // NSUB
    n_chunks = per_w // chunk
    assert n_chunks % 2 == 0 and n_chunks >= 2
    krow = chunk // 128
    rows_per_tile = NLOC // NSUB
    mesh = plsc.VectorSubcoreMesh(core_axis_name="c", subcore_axis_name="s")

    @functools.partial(
        pl.kernel,
        out_type=jax.ShapeDtypeStruct((NCORES, NLOC, D), jnp.float32),
        mesh=mesh,
        scratch_types=[
            pltpu.VMEM((2 * krow, 128), jnp.int32),
            pltpu.VMEM((2 * krow, 128), jnp.int32),
            pltpu.VMEM((chunk, D), jnp.float32),
            pltpu.VMEM((chunk, D), jnp.float32),
            pltpu.VMEM_SHARED((NLOC, D), jnp.float32),
            pltpu.SemaphoreType.DMA,
            pltpu.SemaphoreType.DMA,
            [pltpu.SemaphoreType.DMA] * 4,
            [pltpu.SemaphoreType.DMA] * 4,
        ],
        compiler_params=pltpu.CompilerParams(use_tc_tiling_on_sc=False),
    )
    def segsum(y_hbm, eidx_hbm, zeros_hbm, out_hbm,
               idx0_v, idx1_v, rows0_v, rows1_v, acc_s,
               gsem0, gsem1, ssem0, ssem1):
        cid = lax.axis_index("c")
        sid = lax.axis_index("s")
        tile_base = sid * rows_per_tile
        base_g = sid * n_chunks * 2 * krow

        def fire(row0, idx_v, rows_v, sem):
            pltpu.sync_copy(eidx_hbm.at[cid].at[pl.ds(row0, 2 * krow)], idx_v)
            for j in range(krow):
                pltpu.async_copy(y_hbm.at[idx_v.at[j]],
                                 rows_v.at[pl.ds(j * 128, 128)], sem)

        def drain(rows_v, sem):
            pltpu.make_async_copy(y_hbm.at[pl.ds(0, chunk)], rows_v, sem).wait()

        def scatter(idx_v, rows_v, ssem):
            for j in range(krow):
                pltpu.async_copy(rows_v.at[pl.ds(j * 128, 128)],
                                 acc_s.at[idx_v.at[krow + j]], ssem[j],
                                 add=True)

        def drain_s(idx_v, rows_v, ssem):
            for j in range(krow):
                pltpu.make_async_copy(rows_v.at[pl.ds(j * 128, 128)],
                                      acc_s.at[idx_v.at[krow + j]],
                                      ssem[j]).wait()

        fire(base_g, idx0_v, rows0_v, gsem0)
        pltpu.sync_copy(zeros_hbm,
                        acc_s.at[pl.ds(tile_base, rows_per_tile)])
        plsc.subcore_barrier()

        @pl.loop(0, n_chunks // 2)
        def _(h):
            ci1 = 2 * h + 1
            ci2 = 2 * h + 2
            drain(rows0_v, gsem0)

            @pl.when(h > 0)
            def _():
                drain_s(idx1_v, rows1_v, ssem1)

            fire(base_g + ci1 * 2 * krow, idx1_v, rows1_v, gsem1)
            scatter(idx0_v, rows0_v, ssem0)
            drain(rows1_v, gsem1)

            @pl.when(ci2 < n_chunks)
            def _():
                drain_s(idx0_v, rows0_v, ssem0)
                fire(base_g + ci2 * 2 * krow, idx0_v, rows0_v, gsem0)

            scatter(idx1_v, rows1_v, ssem1)

        drain_s(idx0_v, rows0_v, ssem0)
        drain_s(idx1_v, rows1_v, ssem1)
        plsc.subcore_barrier()
        pltpu.sync_copy(
            acc_s.at[pl.ds(tile_base, rows_per_tile)],
            out_hbm.at[cid].at[pl.ds(tile_base, rows_per_tile)])

    return segsum


@functools.lru_cache(maxsize=None)
def _make_pairsum(n_pad):
    out_per_w = n_pad // NW
    oc = 128
    n_chunks = out_per_w // oc
    mesh = plsc.VectorSubcoreMesh(core_axis_name="c", subcore_axis_name="s")

    @functools.partial(
        pl.kernel,
        out_type=jax.ShapeDtypeStruct((n_pad, D), jnp.float32),
        mesh=mesh,
        scratch_types=[
            pltpu.VMEM((2, 128), jnp.int32),
            pltpu.VMEM((2, 128), jnp.int32),
            pltpu.VMEM((2 * oc, D), jnp.float32),
            pltpu.VMEM((2 * oc, D), jnp.float32),
            pltpu.VMEM((oc, D), jnp.float32),
            pltpu.VMEM((oc, D), jnp.float32),
            pltpu.SemaphoreType.DMA,
            pltpu.SemaphoreType.DMA,
            pltpu.SemaphoreType.DMA,
            pltpu.SemaphoreType.DMA,
        ],
        compiler_params=pltpu.CompilerParams(use_tc_tiling_on_sc=False),
    )
    def pairsum(y_hbm, src_hbm, out_hbm,
                idx0_v, idx1_v, rows0_v, rows1_v, out0_v, out1_v,
                gsem0, gsem1, osem0, osem1):
        cid = lax.axis_index("c")
        sid = lax.axis_index("s")
        wid = cid * NSUB + sid
        base_out = wid * out_per_w
        base_idx = wid * n_chunks * 2

        def fire(ci, idx_v, rows_v, sem):
            pltpu.sync_copy(src_hbm.at[pl.ds(base_idx + ci * 2, 2)], idx_v)
            for j in range(2):
                pltpu.async_copy(y_hbm.at[idx_v.at[j]],
                                 rows_v.at[pl.ds(j * 128, 128)], sem)

        def drain_g(rows_v, sem):
            pltpu.make_async_copy(y_hbm.at[pl.ds(0, 2 * oc)], rows_v, sem).wait()

        def reduce_write(ci, rows_v, out_v, sem):
            @pl.loop(0, oc)
            def _(i):
                for j in range(D // 16):
                    s = pl.ds(j * 16, 16)
                    out_v[i, s] = rows_v[2 * i, s] + rows_v[2 * i + 1, s]
            pltpu.async_copy(out_v, out_hbm.at[pl.ds(base_out + ci * oc, oc)],
                             sem)

        def drain_o(out_v, sem):
            pltpu.make_async_copy(y_hbm.at[pl.ds(0, oc)], out_v, sem).wait()

        fire(0, idx0_v, rows0_v, gsem0)

        @pl.loop(0, (n_chunks + 1) // 2)
        def _(h):
            ci0 = 2 * h
            ci1 = 2 * h + 1
            ci2 = 2 * h + 2
            drain_g(rows0_v, gsem0)

            @pl.when(ci1 < n_chunks)
            def _():
                fire(ci1, idx1_v, rows1_v, gsem1)

            @pl.when(h > 0)
            def _():
                drain_o(out0_v, osem0)

            reduce_write(ci0, rows0_v, out0_v, osem0)

            @pl.when(ci1 < n_chunks)
            def _():
                drain_g(rows1_v, gsem1)

                @pl.when(ci2 < n_chunks)
                def _():
                    fire(ci2, idx0_v, rows0_v, gsem0)

                @pl.when(h > 0)
                def _():
                    drain_o(out1_v, osem1)

                reduce_write(ci1, rows1_v, out1_v, osem1)

        drain_o(out0_v, osem0)
        if n_chunks > 1:
            drain_o(out1_v, osem1)

    return pairsum


_BS = 1024


def _mm(x, w, b):
    n, din = x.shape
    dout = w.shape[1]

    def body(x_ref, w_ref, b_ref, o_ref):
        o_ref[...] = jnp.dot(x_ref[...], w_ref[...],
                             preferred_element_type=jnp.float32) + b_ref[...]

    return pl.pallas_call(
        body,
        grid=(n // _BS,),
        in_specs=[
            pl.BlockSpec((_BS, din), lambda i: (i, 0)),
            pl.BlockSpec((din, dout), lambda i: (0, 0)),
            pl.BlockSpec((1, dout), lambda i: (0, 0)),
        ],
        out_specs=pl.BlockSpec((_BS, dout), lambda i: (i, 0)),
        out_shape=jax.ShapeDtypeStruct((n, dout), jnp.float32),
    )(x, w, b.reshape(1, -1))


def _agg_spec(split):
    if split:
        nh = NHALF // _BS
        return pl.BlockSpec((1, _BS, D), lambda i: (i // nh, i % nh, 0))
    return pl.BlockSpec((NCORES, _BS, D), lambda i: (0, i, 0))


def _comb_mm(aggp, r, w, b, split=False):
    n = r.shape[0]
    dout = w.shape[1]

    def body(a_ref, r_ref, w_ref, b_ref, o_ref):
        agg = a_ref[0] if split else a_ref[0] + a_ref[1]
        h = jnp.maximum(agg + r_ref[...], 0.0)
        o_ref[...] = jnp.dot(h, w_ref[...],
                             preferred_element_type=jnp.float32) + b_ref[...]

    return pl.pallas_call(
        body,
        grid=(n // _BS,),
        in_specs=[
            _agg_spec(split),
            pl.BlockSpec((_BS, D), lambda i: (i, 0)),
            pl.BlockSpec((D, dout), lambda i: (0, 0)),
            pl.BlockSpec((1, dout), lambda i: (0, 0)),
        ],
        out_specs=pl.BlockSpec((_BS, dout), lambda i: (i, 0)),
        out_shape=jax.ShapeDtypeStruct((n, dout), jnp.float32),
    )(aggp, r, w, b.reshape(1, -1))


def _comb_pool(aggp, r, seg3d, split=False):
    n = r.shape[0]

    def body(a_ref, r_ref, s_ref, h_ref, sum_ref, cnt_ref):
        i = pl.program_id(0)
        agg = a_ref[0] if split else a_ref[0] + a_ref[1]
        h = jnp.maximum(agg + r_ref[...], 0.0)
        h_ref[...] = h
        ids = s_ref[0, 0, :]
        mask = (lax.broadcasted_iota(jnp.int32, (G, _BS), 0)
                == ids[None, :]).astype(jnp.float32)

        @pl.when(i == 0)
        def _():
            sum_ref[...] = jnp.zeros_like(sum_ref)
            cnt_ref[...] = jnp.zeros_like(cnt_ref)

        sum_ref[...] += jnp.dot(mask, h,
                                preferred_element_type=jnp.float32)
        cnt_ref[...] += jnp.sum(mask, axis=1, keepdims=True)

    return pl.pallas_call(
        body,
        grid=(n // _BS,),
        in_specs=[
            _agg_spec(split),
            pl.BlockSpec((_BS, D), lambda i: (i, 0)),
            pl.BlockSpec((1, 1, _BS), lambda i: (i, 0, 0)),
        ],
        out_specs=[
            pl.BlockSpec((_BS, D), lambda i: (i, 0)),
            pl.BlockSpec((G, D), lambda i: (0, 0)),
            pl.BlockSpec((G, 1), lambda i: (0, 0)),
        ],
        out_shape=[
            jax.ShapeDtypeStruct((n, D), jnp.float32),
            jax.ShapeDtypeStruct((G, D), jnp.float32),
            jax.ShapeDtypeStruct((G, 1), jnp.float32),
        ],
    )(aggp, r, seg3d)


def _tuple_mm(tp, iso, wa, wb, b):
    n = iso.shape[0]
    dout = wa.shape[1]

    def body(t_ref, i_ref, wa_ref, wb_ref, b_ref, o_ref):
        h = t_ref[...] * 0.5
        o_ref[...] = (jnp.dot(h, wa_ref[...], preferred_element_type=jnp.float32)
                      + jnp.dot(i_ref[...], wb_ref[...],
                                preferred_element_type=jnp.float32)
                      + b_ref[...])

    return pl.pallas_call(
        body,
        grid=(n // _BS,),
        in_specs=[
            pl.BlockSpec((_BS, D), lambda i: (i, 0)),
            pl.BlockSpec((_BS, NI2), lambda i: (i, 0)),
            pl.BlockSpec((D, dout), lambda i: (0, 0)),
            pl.BlockSpec((NI2, dout), lambda i: (0, 0)),
            pl.BlockSpec((1, dout), lambda i: (0, 0)),
        ],
        out_specs=pl.BlockSpec((_BS, dout), lambda i: (i, 0)),
        out_shape=jax.ShapeDtypeStruct((n, dout), jnp.float32),
    )(tp, iso, wa, wb, b.reshape(1, -1))


def _pool(h, seg3d):
    n = h.shape[0]
    nblk = n // _BS

    def body(h_ref, s_ref, sum_ref, cnt_ref):
        i = pl.program_id(0)
        ids = s_ref[0, 0, :]
        mask = (lax.broadcasted_iota(jnp.int32, (G, _BS), 0)
                == ids[None, :]).astype(jnp.float32)

        @pl.when(i == 0)
        def _():
            sum_ref[...] = jnp.zeros_like(sum_ref)
            cnt_ref[...] = jnp.zeros_like(cnt_ref)

        sum_ref[...] += jnp.dot(mask, h_ref[...],
                                preferred_element_type=jnp.float32)
        cnt_ref[...] += jnp.sum(mask, axis=1, keepdims=True)

    return pl.pallas_call(
        body,
        grid=(nblk,),
        in_specs=[
            pl.BlockSpec((_BS, D), lambda i: (i, 0)),
            pl.BlockSpec((1, 1, _BS), lambda i: (i, 0, 0)),
        ],
        out_specs=[
            pl.BlockSpec((G, D), lambda i: (0, 0)),
            pl.BlockSpec((G, 1), lambda i: (0, 0)),
        ],
        out_shape=[
            jax.ShapeDtypeStruct((G, D), jnp.float32),
            jax.ShapeDtypeStruct((G, 1), jnp.float32),
        ],
    )(h, seg3d)


def _final_mlp(s0, c0, s1, c1, wfc0, bfc0, wfc1, bfc1, wfc2, bfc2):
    def body(s0_ref, c0_ref, s1_ref, c1_ref, w0_ref, b0_ref, w1_ref, b1_ref,
             w2_ref, b2_ref, o_ref):
        x0 = s0_ref[...] / jnp.maximum(c0_ref[...], 1.0)
        x1 = s1_ref[...] / jnp.maximum(c1_ref[...], 1.0)
        a = jnp.maximum(
            jnp.dot(x0, w0_ref[:D], preferred_element_type=jnp.float32)
            + jnp.dot(x1, w0_ref[D:], preferred_element_type=jnp.float32)
            + b0_ref[...], 0.0)
        h = jnp.maximum(
            jnp.dot(a, w1_ref[...], preferred_element_type=jnp.float32)
            + b1_ref[...], 0.0)
        o = jnp.dot(h, w2_ref[...], preferred_element_type=jnp.float32) + b2_ref[...]
        m = jnp.max(o, axis=1, keepdims=True)
        z = o - m
        lse = jnp.log(jnp.sum(jnp.exp(z), axis=1, keepdims=True))
        o_ref[...] = z - lse

    return pl.pallas_call(
        body,
        out_shape=jax.ShapeDtypeStruct((G, NC), jnp.float32),
    )(s0, c0, s1, c1, wfc0, bfc0.reshape(1, -1), wfc1, bfc1.reshape(1, -1),
      wfc2, bfc2.reshape(1, -1))


def _pad_rows(x, n_pad, fill=0.0):
    return jnp.pad(x, ((0, n_pad - x.shape[0]), (0, 0)), constant_values=fill)


def _prep_edges(src, dst, e_pad, dummy_dst, chunk):
    e = src.shape[0]
    krow = chunk // 128
    src = jnp.pad(src.astype(jnp.int32), (0, e_pad - e),
                  constant_values=0).reshape(e_pad // chunk, krow, 128)
    dst = jnp.pad(dst.astype(jnp.int32), (0, e_pad - e),
                  constant_values=dummy_dst).reshape(e_pad // chunk, krow, 128)
    return jnp.concatenate([src, dst], axis=1).reshape(-1, 128)


def _prep_edges_split(src, dst, e_pad, chunk):
    e = src.shape[0]
    krow = chunk // 128
    src = jnp.pad(src.astype(jnp.int32), (0, e_pad - e), constant_values=0)
    dst = jnp.pad(dst.astype(jnp.int32), (0, e_pad - e),
                  constant_values=2 * NHALF)
    d0 = jnp.where(dst < NHALF, dst, NLOC - 1)
    d1 = jnp.where(dst >= NHALF, dst - NHALF, NLOC - 1)
    out = []
    for d in (d0, d1):
        sc = src.reshape(e_pad // chunk, krow, 128)
        dc = d.reshape(e_pad // chunk, krow, 128)
        out.append(jnp.concatenate([sc, dc], axis=1).reshape(-1, 128))
    return jnp.stack(out)


def kernel(x, edge_index, edge_index_2, batch, batch_2, assignment_index_2,
           iso_type_2, W_init_rel, b_init, W_init_root,
           W00_rel, b00, W00_root, W01_rel, b01, W01_root,
           W10_rel, b10, W10_root, W11_rel, b11, W11_root,
           Wfc0, bfc0, Wfc1, bfc1, Wfc2, bfc2):
    zeros1 = jnp.zeros((NPAD // NSUB, D), jnp.float32)
    zeros2 = jnp.zeros((NLOC // NSUB, D), jnp.float32)

    eidx1 = _prep_edges(edge_index[0], edge_index[1], EPAD, N, 512)
    srca = jnp.pad(assignment_index_2[0].astype(jnp.int32),
                   (0, 2 * N2PAD - A), constant_values=0).reshape(-1, 128)
    eidx2 = _prep_edges_split(edge_index_2[0], edge_index_2[1], E2PAD, 512)

    batch3d = jnp.pad(batch.astype(jnp.int32), (0, NPAD - N),
                      constant_values=G).reshape(NPAD // _BS, 1, _BS)
    batch23d = jnp.pad(batch_2.astype(jnp.int32), (0, N2PAD - N2),
                       constant_values=G).reshape(N2PAD // _BS, 1, _BS)

    xp = _pad_rows(x, NPAD)
    isop = _pad_rows(iso_type_2, N2PAD)

    seg1 = _make_segsum(NPAD, EPAD, 512)
    pairsum = _make_pairsum(N2PAD)
    seg2 = _make_segsum_split(E2PAD, 512)

    w_init = jnp.concatenate([W_init_rel, W_init_root], axis=1)
    b_init_c = jnp.concatenate([jnp.zeros((D,), jnp.float32), b_init])
    w00 = jnp.concatenate([W00_rel, W00_root], axis=1)
    b00_c = jnp.concatenate([jnp.zeros((D,), jnp.float32), b00])
    w01 = jnp.concatenate([W01_rel, W01_root], axis=1)
    b01_c = jnp.concatenate([jnp.zeros((D,), jnp.float32), b01])
    w10a = jnp.concatenate([W10_rel[:D], W10_root[:D]], axis=1)
    w10b = jnp.concatenate([W10_rel[D:], W10_root[D:]], axis=1)
    b10_c = jnp.concatenate([jnp.zeros((D,), jnp.float32), b10])
    w11 = jnp.concatenate([W11_rel, W11_root], axis=1)
    b11_c = jnp.concatenate([jnp.zeros((D,), jnp.float32), b11])

    yr0 = _mm(xp, w_init, b_init_c)
    y0, r0 = yr0[:, :D], yr0[:, D:]
    agg0 = seg1(y0, eidx1, zeros1)

    yr1 = _comb_mm(agg0, r0, w00, b00_c)
    y1, r1 = yr1[:, :D], yr1[:, D:]
    agg1 = seg1(y1, eidx1, zeros1)

    yr2 = _comb_mm(agg1, r1, w01, b01_c)
    y2, r2 = yr2[:, :D], yr2[:, D:]
    agg2 = seg1(y2, eidx1, zeros1)

    h3, s0, c0 = _comb_pool(agg2, r2, batch3d)

    tp = pairsum(h3, srca)
    yr4 = _tuple_mm(tp, isop, w10a, w10b, b10_c)
    y4, r4 = yr4[:, :D], yr4[:, D:]
    agg3 = seg2(y4, eidx2, zeros2)

    yr5 = _comb_mm(agg3, r4, w11, b11_c, split=True)
    y5, r5 = yr5[:, :D], yr5[:, D:]
    agg4 = seg2(y5, eidx2, zeros2)

    _, s1, c1 = _comb_pool(agg4, r5, batch23d, split=True)

    return _final_mlp(s0, c0, s1, c1, Wfc0, bfc0, Wfc1, bfc1, Wfc2, bfc2)


# speedup vs baseline: 1.4157x; 1.4157x over previous
import functools

import jax
import jax.numpy as jnp
from jax import lax
from jax.experimental import pallas as pl
from jax.experimental.pallas import tpu as pltpu
from jax.experimental.pallas import tpu_sc as plsc

N = 10000
E = 320000
N2 = 20000
E2 = 640000
A = 40000
G = 256
D_FEAT = 128
D = 64
NI2 = 16
NC = 10

NCORES = 2
NSUB = 16
NW = NCORES * NSUB

NPAD = 10240
N2PAD = 20480
EPAD = 327680
APAD = 49152
E2PAD = 655360

@functools.lru_cache(maxsize=None)
def _make_segsum(n_pad, e_pad, chunk, async_scatter, small_zero):
    per_w = e_pad // NW
    n_chunks = per_w // chunk
    assert n_chunks % 2 == 0 and n_chunks >= 2
    krow = chunk // 128
    rows_per_tile = n_pad // NSUB
    mesh = plsc.VectorSubcoreMesh(core_axis_name="c", subcore_axis_name="s")

    @functools.partial(
        pl.kernel,
        out_type=jax.ShapeDtypeStruct((NCORES, n_pad, D), jnp.float32),
        mesh=mesh,
        scratch_types=[
            pltpu.VMEM((2 * krow, 128), jnp.int32),
            pltpu.VMEM((2 * krow, 128), jnp.int32),
            pltpu.VMEM((chunk, D), jnp.float32),
            pltpu.VMEM((chunk, D), jnp.float32),
            pltpu.VMEM_SHARED((n_pad, D), jnp.float32),
            pltpu.SemaphoreType.DMA,
            pltpu.SemaphoreType.DMA,
            [pltpu.SemaphoreType.DMA] * 4,
            [pltpu.SemaphoreType.DMA] * 4,
        ],
        compiler_params=pltpu.CompilerParams(use_tc_tiling_on_sc=False),
    )
    def segsum(y_hbm, eidx_hbm, zeros_hbm, out_hbm,
               idx0_v, idx1_v, rows0_v, rows1_v, acc_s,
               gsem0, gsem1, ssem0, ssem1):
        cid = lax.axis_index("c")
        sid = lax.axis_index("s")
        wid = cid * NSUB + sid
        tile_base = sid * rows_per_tile
        base_g = wid * n_chunks * 2 * krow

        def fire(row0, idx_v, rows_v, sem):
            pltpu.sync_copy(eidx_hbm.at[pl.ds(row0, 2 * krow)], idx_v)
            for j in range(krow):
                pltpu.async_copy(y_hbm.at[idx_v.at[j]],
                                 rows_v.at[pl.ds(j * 128, 128)], sem)

        def drain(rows_v, sem):
            pltpu.make_async_copy(y_hbm.at[pl.ds(0, chunk)], rows_v, sem).wait()

        def scatter(idx_v, rows_v, ssem):
            for j in range(krow):
                if async_scatter:
                    pltpu.async_copy(rows_v.at[pl.ds(j * 128, 128)],
                                     acc_s.at[idx_v.at[krow + j]], ssem[j],
                                     add=True)
                else:
                    pltpu.sync_copy(rows_v.at[pl.ds(j * 128, 128)],
                                    acc_s.at[idx_v.at[krow + j]], add=True)

        def drain_s(idx_v, rows_v, ssem):
            for j in range(krow):
                pltpu.make_async_copy(rows_v.at[pl.ds(j * 128, 128)],
                                      acc_s.at[idx_v.at[krow + j]],
                                      ssem[j]).wait()

        fire(base_g, idx0_v, rows0_v, gsem0)
        if small_zero:
            @pl.loop(0, rows_per_tile // 128)
            def _(b):
                pltpu.sync_copy(zeros_hbm,
                                acc_s.at[pl.ds(tile_base + b * 128, 128)])
        else:
            pltpu.sync_copy(zeros_hbm,
                            acc_s.at[pl.ds(tile_base, rows_per_tile)])
        plsc.subcore_barrier()

        @pl.loop(0, n_chunks // 2)
        def _(h):
            ci1 = 2 * h + 1
            ci2 = 2 * h + 2
            drain(rows0_v, gsem0)

            if async_scatter:
                @pl.when(h > 0)
                def _():
                    drain_s(idx1_v, rows1_v, ssem1)

            fire(base_g + ci1 * 2 * krow, idx1_v, rows1_v, gsem1)
            scatter(idx0_v, rows0_v, ssem0)
            drain(rows1_v, gsem1)

            @pl.when(ci2 < n_chunks)
            def _():
                if async_scatter:
                    drain_s(idx0_v, rows0_v, ssem0)
                fire(base_g + ci2 * 2 * krow, idx0_v, rows0_v, gsem0)

            scatter(idx1_v, rows1_v, ssem1)

        if async_scatter:
            drain_s(idx0_v, rows0_v, ssem0)
            drain_s(idx1_v, rows1_v, ssem1)
        plsc.subcore_barrier()
        pltpu.sync_copy(
            acc_s.at[pl.ds(tile_base, rows_per_tile)],
            out_hbm.at[cid].at[pl.ds(tile_base, rows_per_tile)])

    return segsum


NHALF = 10240
NLOC = 10368


@functools.lru_cache(maxsize=None)
def _make_segsum_split(e_pad, chunk):
    per_w = e_pad // NSUB
    n_chunks = per_w // chunk
    assert n_chunks % 2 == 0 and n_chunks >= 2
    krow = chunk // 128
    rows_per_tile = NLOC // NSUB
    mesh = plsc.VectorSubcoreMesh(core_axis_name="c", subcore_axis_name="s")

    @functools.partial(
        pl.kernel,
        out_type=jax.ShapeDtypeStruct((NCORES, NLOC, D), jnp.float32),
        mesh=mesh,
        scratch_types=[
            pltpu.VMEM((2 * krow, 128), jnp.int32),
            pltpu.VMEM((2 * krow, 128), jnp.int32),
            pltpu.VMEM((chunk, D), jnp.float32),
            pltpu.VMEM((chunk, D), jnp.float32),
            pltpu.VMEM_SHARED((NLOC, D), jnp.float32),
            pltpu.SemaphoreType.DMA,
            pltpu.SemaphoreType.DMA,
            [pltpu.SemaphoreType.DMA] * 4,
            [pltpu.SemaphoreType.DMA] * 4,
        ],
        compiler_params=pltpu.CompilerParams(use_tc_tiling_on_sc=False),
    )
    def segsum(y_hbm, eidx_hbm, zeros_hbm, out_hbm,
               idx0_v, idx1_v, rows0_v, rows1_v, acc_s,
               gsem0, gsem1, ssem0, ssem1):
        cid = lax.axis_index("c")
        sid = lax.axis_index("s")
        tile_base = sid * rows_per_tile
        base_g = sid * n_chunks * 2 * krow

        def fire(row0, idx_v, rows_v, sem):
            pltpu.sync_copy(eidx_hbm.at[cid].at[pl.ds(row0, 2 * krow)], idx_v)
            for j in range(krow):
                pltpu.async_copy(y_hbm.at[idx_v.at[j]],
                                 rows_v.at[pl.ds(j * 128, 128)], sem)

        def drain(rows_v, sem):
            pltpu.make_async_copy(y_hbm.at[pl.ds(0, chunk)], rows_v, sem).wait()

        def scatter(idx_v, rows_v, ssem):
            for j in range(krow):
                pltpu.async_copy(rows_v.at[pl.ds(j * 128, 128)],
                                 acc_s.at[idx_v.at[krow + j]], ssem[j],
                                 add=True)

        def drain_s(idx_v, rows_v, ssem):
            for j in range(krow):
                pltpu.make_async_copy(rows_v.at[pl.ds(j * 128, 128)],
                                      acc_s.at[idx_v.at[krow + j]],
                                      ssem[j]).wait()

        fire(base_g, idx0_v, rows0_v, gsem0)
        pltpu.sync_copy(zeros_hbm,
                        acc_s.at[pl.ds(tile_base, rows_per_tile)])
        plsc.subcore_barrier()

        @pl.loop(0, n_chunks // 2)
        def _(h):
            ci1 = 2 * h + 1
            ci2 = 2 * h + 2
            drain(rows0_v, gsem0)

            @pl.when(h > 0)
            def _():
                drain_s(idx1_v, rows1_v, ssem1)

            fire(base_g + ci1 * 2 * krow, idx1_v, rows1_v, gsem1)
            scatter(idx0_v, rows0_v, ssem0)
            drain(rows1_v, gsem1)

            @pl.when(ci2 < n_chunks)
            def _():
                drain_s(idx0_v, rows0_v, ssem0)
                fire(base_g + ci2 * 2 * krow, idx0_v, rows0_v, gsem0)

            scatter(idx1_v, rows1_v, ssem1)

        drain_s(idx0_v, rows0_v, ssem0)
        drain_s(idx1_v, rows1_v, ssem1)
        plsc.subcore_barrier()
        pltpu.sync_copy(
            acc_s.at[pl.ds(tile_base, rows_per_tile)],
            out_hbm.at[cid].at[pl.ds(tile_base, rows_per_tile)])

    return segsum


@functools.lru_cache(maxsize=None)
def _make_pairsum(n_pad):
    out_per_w = n_pad // NW
    oc = 128
    n_chunks = out_per_w // oc
    mesh = plsc.VectorSubcoreMesh(core_axis_name="c", subcore_axis_name="s")

    @functools.partial(
        pl.kernel,
        out_type=jax.ShapeDtypeStruct((n_pad, D), jnp.float32),
        mesh=mesh,
        scratch_types=[
            pltpu.VMEM((2, 128), jnp.int32),
            pltpu.VMEM((2, 128), jnp.int32),
            pltpu.VMEM((2 * oc, D), jnp.float32),
            pltpu.VMEM((2 * oc, D), jnp.float32),
            pltpu.VMEM((oc, D), jnp.float32),
            pltpu.VMEM((oc, D), jnp.float32),
            pltpu.SemaphoreType.DMA,
            pltpu.SemaphoreType.DMA,
            pltpu.SemaphoreType.DMA,
            pltpu.SemaphoreType.DMA,
        ],
        compiler_params=pltpu.CompilerParams(use_tc_tiling_on_sc=False),
    )
    def pairsum(y_hbm, src_hbm, out_hbm,
                idx0_v, idx1_v, rows0_v, rows1_v, out0_v, out1_v,
                gsem0, gsem1, osem0, osem1):
        cid = lax.axis_index("c")
        sid = lax.axis_index("s")
        wid = cid * NSUB + sid
        base_out = wid * out_per_w
        base_idx = wid * n_chunks * 2

        def fire(ci, idx_v, rows_v, sem):
            pltpu.sync_copy(src_hbm.at[pl.ds(base_idx + ci * 2, 2)], idx_v)
            for j in range(2):
                pltpu.async_copy(y_hbm.at[idx_v.at[j]],
                                 rows_v.at[pl.ds(j * 128, 128)], sem)

        def drain_g(rows_v, sem):
            pltpu.make_async_copy(y_hbm.at[pl.ds(0, 2 * oc)], rows_v, sem).wait()

        def reduce_write(ci, rows_v, out_v, sem):
            @pl.loop(0, oc)
            def _(i):
                for j in range(D // 16):
                    s = pl.ds(j * 16, 16)
                    out_v[i, s] = rows_v[2 * i, s] + rows_v[2 * i + 1, s]
            pltpu.async_copy(out_v, out_hbm.at[pl.ds(base_out + ci * oc, oc)],
                             sem)

        def drain_o(out_v, sem):
            pltpu.make_async_copy(y_hbm.at[pl.ds(0, oc)], out_v, sem).wait()

        fire(0, idx0_v, rows0_v, gsem0)

        @pl.loop(0, (n_chunks + 1) // 2)
        def _(h):
            ci0 = 2 * h
            ci1 = 2 * h + 1
            ci2 = 2 * h + 2
            drain_g(rows0_v, gsem0)

            @pl.when(ci1 < n_chunks)
            def _():
                fire(ci1, idx1_v, rows1_v, gsem1)

            @pl.when(h > 0)
            def _():
                drain_o(out0_v, osem0)

            reduce_write(ci0, rows0_v, out0_v, osem0)

            @pl.when(ci1 < n_chunks)
            def _():
                drain_g(rows1_v, gsem1)

                @pl.when(ci2 < n_chunks)
                def _():
                    fire(ci2, idx0_v, rows0_v, gsem0)

                @pl.when(h > 0)
                def _():
                    drain_o(out1_v, osem1)

                reduce_write(ci1, rows1_v, out1_v, osem1)

        drain_o(out0_v, osem0)
        if n_chunks > 1:
            drain_o(out1_v, osem1)

    return pairsum


_BS = 1024


def _mm(x, w, b):
    n, din = x.shape
    dout = w.shape[1]

    def body(x_ref, w_ref, b_ref, o_ref):
        o_ref[...] = jnp.dot(x_ref[...], w_ref[...],
                             preferred_element_type=jnp.float32) + b_ref[...]

    return pl.pallas_call(
        body,
        grid=(n // _BS,),
        in_specs=[
            pl.BlockSpec((_BS, din), lambda i: (i, 0)),
            pl.BlockSpec((din, dout), lambda i: (0, 0)),
            pl.BlockSpec((1, dout), lambda i: (0, 0)),
        ],
        out_specs=pl.BlockSpec((_BS, dout), lambda i: (i, 0)),
        out_shape=jax.ShapeDtypeStruct((n, dout), jnp.float32),
    )(x, w, b.reshape(1, -1))


def _agg_spec(split):
    if split:
        nh = NHALF // _BS
        return pl.BlockSpec((1, _BS, D), lambda i: (i // nh, i % nh, 0))
    return pl.BlockSpec((NCORES, _BS, D), lambda i: (0, i, 0))


def _comb_mm(aggp, r, w, b, split=False):
    n = r.shape[0]
    dout = w.shape[1]

    def body(a_ref, r_ref, w_ref, b_ref, o_ref):
        agg = a_ref[0] if split else a_ref[0] + a_ref[1]
        h = jnp.maximum(agg + r_ref[...], 0.0)
        o_ref[...] = jnp.dot(h, w_ref[...],
                             preferred_element_type=jnp.float32) + b_ref[...]

    return pl.pallas_call(
        body,
        grid=(n // _BS,),
        in_specs=[
            _agg_spec(split),
            pl.BlockSpec((_BS, D), lambda i: (i, 0)),
            pl.BlockSpec((D, dout), lambda i: (0, 0)),
            pl.BlockSpec((1, dout), lambda i: (0, 0)),
        ],
        out_specs=pl.BlockSpec((_BS, dout), lambda i: (i, 0)),
        out_shape=jax.ShapeDtypeStruct((n, dout), jnp.float32),
    )(aggp, r, w, b.reshape(1, -1))


def _comb_pool(aggp, r, seg3d, split=False):
    n = r.shape[0]

    def body(a_ref, r_ref, s_ref, h_ref, sum_ref, cnt_ref):
        i = pl.program_id(0)
        agg = a_ref[0] if split else a_ref[0] + a_ref[1]
        h = jnp.maximum(agg + r_ref[...], 0.0)
        h_ref[...] = h
        ids = s_ref[0, 0, :]
        mask = (lax.broadcasted_iota(jnp.int32, (G, _BS), 0)
                == ids[None, :]).astype(jnp.float32)

        @pl.when(i == 0)
        def _():
            sum_ref[...] = jnp.zeros_like(sum_ref)
            cnt_ref[...] = jnp.zeros_like(cnt_ref)

        sum_ref[...] += jnp.dot(mask, h,
                                preferred_element_type=jnp.float32)
        cnt_ref[...] += jnp.sum(mask, axis=1, keepdims=True)

    return pl.pallas_call(
        body,
        grid=(n // _BS,),
        in_specs=[
            _agg_spec(split),
            pl.BlockSpec((_BS, D), lambda i: (i, 0)),
            pl.BlockSpec((1, 1, _BS), lambda i: (i, 0, 0)),
        ],
        out_specs=[
            pl.BlockSpec((_BS, D), lambda i: (i, 0)),
            pl.BlockSpec((G, D), lambda i: (0, 0)),
            pl.BlockSpec((G, 1), lambda i: (0, 0)),
        ],
        out_shape=[
            jax.ShapeDtypeStruct((n, D), jnp.float32),
            jax.ShapeDtypeStruct((G, D), jnp.float32),
            jax.ShapeDtypeStruct((G, 1), jnp.float32),
        ],
    )(aggp, r, seg3d)


def _tuple_mm(tp, iso, wa, wb, b):
    n = iso.shape[0]
    dout = wa.shape[1]

    def body(t_ref, i_ref, wa_ref, wb_ref, b_ref, o_ref):
        h = t_ref[...] * 0.5
        o_ref[...] = (jnp.dot(h, wa_ref[...], preferred_element_type=jnp.float32)
                      + jnp.dot(i_ref[...], wb_ref[...],
                                preferred_element_type=jnp.float32)
                      + b_ref[...])

    return pl.pallas_call(
        body,
        grid=(n // _BS,),
        in_specs=[
            pl.BlockSpec((_BS, D), lambda i: (i, 0)),
            pl.BlockSpec((_BS, NI2), lambda i: (i, 0)),
            pl.BlockSpec((D, dout), lambda i: (0, 0)),
            pl.BlockSpec((NI2, dout), lambda i: (0, 0)),
            pl.BlockSpec((1, dout), lambda i: (0, 0)),
        ],
        out_specs=pl.BlockSpec((_BS, dout), lambda i: (i, 0)),
        out_shape=jax.ShapeDtypeStruct((n, dout), jnp.float32),
    )(tp, iso, wa, wb, b.reshape(1, -1))


def _pool(h, seg3d):
    n = h.shape[0]
    nblk = n // _BS

    def body(h_ref, s_ref, sum_ref, cnt_ref):
        i = pl.program_id(0)
        ids = s_ref[0, 0, :]
        mask = (lax.broadcasted_iota(jnp.int32, (G, _BS), 0)
                == ids[None, :]).astype(jnp.float32)

        @pl.when(i == 0)
        def _():
            sum_ref[...] = jnp.zeros_like(sum_ref)
            cnt_ref[...] = jnp.zeros_like(cnt_ref)

        sum_ref[...] += jnp.dot(mask, h_ref[...],
                                preferred_element_type=jnp.float32)
        cnt_ref[...] += jnp.sum(mask, axis=1, keepdims=True)

    return pl.pallas_call(
        body,
        grid=(nblk,),
        in_specs=[
            pl.BlockSpec((_BS, D), lambda i: (i, 0)),
            pl.BlockSpec((1, 1, _BS), lambda i: (i, 0, 0)),
        ],
        out_specs=[
            pl.BlockSpec((G, D), lambda i: (0, 0)),
            pl.BlockSpec((G, 1), lambda i: (0, 0)),
        ],
        out_shape=[
            jax.ShapeDtypeStruct((G, D), jnp.float32),
            jax.ShapeDtypeStruct((G, 1), jnp.float32),
        ],
    )(h, seg3d)


def _final_mlp(s0, c0, s1, c1, wfc0, bfc0, wfc1, bfc1, wfc2, bfc2):
    def body(s0_ref, c0_ref, s1_ref, c1_ref, w0_ref, b0_ref, w1_ref, b1_ref,
             w2_ref, b2_ref, o_ref):
        x0 = s0_ref[...] / jnp.maximum(c0_ref[...], 1.0)
        x1 = s1_ref[...] / jnp.maximum(c1_ref[...], 1.0)
        a = jnp.maximum(
            jnp.dot(x0, w0_ref[:D], preferred_element_type=jnp.float32)
            + jnp.dot(x1, w0_ref[D:], preferred_element_type=jnp.float32)
            + b0_ref[...], 0.0)
        h = jnp.maximum(
            jnp.dot(a, w1_ref[...], preferred_element_type=jnp.float32)
            + b1_ref[...], 0.0)
        o = jnp.dot(h, w2_ref[...], preferred_element_type=jnp.float32) + b2_ref[...]
        m = jnp.max(o, axis=1, keepdims=True)
        z = o - m
        lse = jnp.log(jnp.sum(jnp.exp(z), axis=1, keepdims=True))
        o_ref[...] = z - lse

    return pl.pallas_call(
        body,
        out_shape=jax.ShapeDtypeStruct((G, NC), jnp.float32),
    )(s0, c0, s1, c1, wfc0, bfc0.reshape(1, -1), wfc1, bfc1.reshape(1, -1),
      wfc2, bfc2.reshape(1, -1))


def _pad_rows(x, n_pad, fill=0.0):
    return jnp.pad(x, ((0, n_pad - x.shape[0]), (0, 0)), constant_values=fill)


def _prep_edges(src, dst, e_pad, dummy_dst, chunk):
    e = src.shape[0]
    krow = chunk // 128
    src = jnp.pad(src.astype(jnp.int32), (0, e_pad - e),
                  constant_values=0).reshape(e_pad // chunk, krow, 128)
    dst = jnp.pad(dst.astype(jnp.int32), (0, e_pad - e),
                  constant_values=dummy_dst).reshape(e_pad // chunk, krow, 128)
    return jnp.concatenate([src, dst], axis=1).reshape(-1, 128)


def _prep_edges_split(src, dst, e_pad, chunk):
    e = src.shape[0]
    krow = chunk // 128
    src = jnp.pad(src.astype(jnp.int32), (0, e_pad - e), constant_values=0)
    dst = jnp.pad(dst.astype(jnp.int32), (0, e_pad - e),
                  constant_values=2 * NHALF)
    d0 = jnp.where(dst < NHALF, dst, NLOC - 1)
    d1 = jnp.where(dst >= NHALF, dst - NHALF, NLOC - 1)
    out = []
    for d in (d0, d1):
        sc = src.reshape(e_pad // chunk, krow, 128)
        dc = d.reshape(e_pad // chunk, krow, 128)
        out.append(jnp.concatenate([sc, dc], axis=1).reshape(-1, 128))
    return jnp.stack(out)


def kernel(x, edge_index, edge_index_2, batch, batch_2, assignment_index_2,
           iso_type_2, W_init_rel, b_init, W_init_root,
           W00_rel, b00, W00_root, W01_rel, b01, W01_root,
           W10_rel, b10, W10_root, W11_rel, b11, W11_root,
           Wfc0, bfc0, Wfc1, bfc1, Wfc2, bfc2):
    zeros1 = jnp.zeros((NPAD // NSUB, D), jnp.float32)
    zeros2 = jnp.zeros((128, D), jnp.float32)

    eidx1 = _prep_edges(edge_index[0], edge_index[1], EPAD, N, 512)
    srca = jnp.pad(assignment_index_2[0].astype(jnp.int32),
                   (0, 2 * N2PAD - A), constant_values=0).reshape(-1, 128)
    eidx2 = _prep_edges(edge_index_2[0], edge_index_2[1], E2PAD, N2, 256)

    batch3d = jnp.pad(batch.astype(jnp.int32), (0, NPAD - N),
                      constant_values=G).reshape(NPAD // _BS, 1, _BS)
    batch23d = jnp.pad(batch_2.astype(jnp.int32), (0, N2PAD - N2),
                       constant_values=G).reshape(N2PAD // _BS, 1, _BS)

    xp = _pad_rows(x, NPAD)
    isop = _pad_rows(iso_type_2, N2PAD)

    seg1 = _make_segsum(NPAD, EPAD, 512, True, False)
    pairsum = _make_pairsum(N2PAD)
    seg2 = _make_segsum(N2PAD, E2PAD, 256, False, True)

    w_init = jnp.concatenate([W_init_rel, W_init_root], axis=1)
    b_init_c = jnp.concatenate([jnp.zeros((D,), jnp.float32), b_init])
    w00 = jnp.concatenate([W00_rel, W00_root], axis=1)
    b00_c = jnp.concatenate([jnp.zeros((D,), jnp.float32), b00])
    w01 = jnp.concatenate([W01_rel, W01_root], axis=1)
    b01_c = jnp.concatenate([jnp.zeros((D,), jnp.float32), b01])
    w10a = jnp.concatenate([W10_rel[:D], W10_root[:D]], axis=1)
    w10b = jnp.concatenate([W10_rel[D:], W10_root[D:]], axis=1)
    b10_c = jnp.concatenate([jnp.zeros((D,), jnp.float32), b10])
    w11 = jnp.concatenate([W11_rel, W11_root], axis=1)
    b11_c = jnp.concatenate([jnp.zeros((D,), jnp.float32), b11])

    yr0 = _mm(xp, w_init, b_init_c)
    y0, r0 = yr0[:, :D], yr0[:, D:]
    agg0 = seg1(y0, eidx1, zeros1)

    yr1 = _comb_mm(agg0, r0, w00, b00_c)
    y1, r1 = yr1[:, :D], yr1[:, D:]
    agg1 = seg1(y1, eidx1, zeros1)

    yr2 = _comb_mm(agg1, r1, w01, b01_c)
    y2, r2 = yr2[:, :D], yr2[:, D:]
    agg2 = seg1(y2, eidx1, zeros1)

    h3, s0, c0 = _comb_pool(agg2, r2, batch3d)

    tp = pairsum(h3, srca)
    yr4 = _tuple_mm(tp, isop, w10a, w10b, b10_c)
    y4, r4 = yr4[:, :D], yr4[:, D:]
    agg3 = seg2(y4, eidx2, zeros2)

    yr5 = _comb_mm(agg3, r4, w11, b11_c)
    y5, r5 = yr5[:, :D], yr5[:, D:]
    agg4 = seg2(y5, eidx2, zeros2)

    _, s1, c1 = _comb_pool(agg4, r5, batch23d)

    return _final_mlp(s0, c0, s1, c1, Wfc0, bfc0, Wfc1, bfc1, Wfc2, bfc2)


# speedup vs baseline: 1.4251x; 1.0067x over previous
import functools

import jax
import jax.numpy as jnp
from jax import lax
from jax.experimental import pallas as pl
from jax.experimental.pallas import tpu as pltpu
from jax.experimental.pallas import tpu_sc as plsc

N = 10000
E = 320000
N2 = 20000
E2 = 640000
A = 40000
G = 256
D_FEAT = 128
D = 64
NI2 = 16
NC = 10

NCORES = 2
NSUB = 16
NW = NCORES * NSUB

NPAD = 10240
N2PAD = 20480
EPAD = 327680
APAD = 49152
E2PAD = 655360

@functools.lru_cache(maxsize=None)
def _make_segsum(n_pad, e_pad, chunk, async_scatter, small_zero):
    per_w = e_pad // NW
    n_chunks = per_w // chunk
    assert n_chunks % 2 == 0 and n_chunks >= 2
    krow = chunk // 128
    rows_per_tile = n_pad // NSUB
    mesh = plsc.VectorSubcoreMesh(core_axis_name="c", subcore_axis_name="s")

    @functools.partial(
        pl.kernel,
        out_type=jax.ShapeDtypeStruct((NCORES, n_pad, D), jnp.float32),
        mesh=mesh,
        scratch_types=[
            pltpu.VMEM((2 * krow, 128), jnp.int32),
            pltpu.VMEM((2 * krow, 128), jnp.int32),
            pltpu.VMEM((chunk, D), jnp.float32),
            pltpu.VMEM((chunk, D), jnp.float32),
            pltpu.VMEM_SHARED((n_pad, D), jnp.float32),
            pltpu.SemaphoreType.DMA,
            pltpu.SemaphoreType.DMA,
            [pltpu.SemaphoreType.DMA] * 4,
            [pltpu.SemaphoreType.DMA] * 4,
        ],
        compiler_params=pltpu.CompilerParams(use_tc_tiling_on_sc=False),
    )
    def segsum(y_hbm, eidx_hbm, zeros_hbm, out_hbm,
               idx0_v, idx1_v, rows0_v, rows1_v, acc_s,
               gsem0, gsem1, ssem0, ssem1):
        cid = lax.axis_index("c")
        sid = lax.axis_index("s")
        wid = cid * NSUB + sid
        tile_base = sid * rows_per_tile
        base_g = wid * n_chunks * 2 * krow

        def fire(row0, idx_v, rows_v, sem):
            pltpu.sync_copy(eidx_hbm.at[pl.ds(row0, 2 * krow)], idx_v)
            for j in range(krow):
                pltpu.async_copy(y_hbm.at[idx_v.at[j]],
                                 rows_v.at[pl.ds(j * 128, 128)], sem)

        def drain(rows_v, sem):
            pltpu.make_async_copy(y_hbm.at[pl.ds(0, chunk)], rows_v, sem).wait()

        def scatter(idx_v, rows_v, ssem):
            for j in range(krow):
                if async_scatter:
                    pltpu.async_copy(rows_v.at[pl.ds(j * 128, 128)],
                                     acc_s.at[idx_v.at[krow + j]], ssem[j],
                                     add=True)
                else:
                    pltpu.sync_copy(rows_v.at[pl.ds(j * 128, 128)],
                                    acc_s.at[idx_v.at[krow + j]], add=True)

        def drain_s(idx_v, rows_v, ssem):
            for j in range(krow):
                pltpu.make_async_copy(rows_v.at[pl.ds(j * 128, 128)],
                                      acc_s.at[idx_v.at[krow + j]],
                                      ssem[j]).wait()

        fire(base_g, idx0_v, rows0_v, gsem0)
        if small_zero:
            @pl.loop(0, rows_per_tile // 128)
            def _(b):
                pltpu.sync_copy(zeros_hbm,
                                acc_s.at[pl.ds(tile_base + b * 128, 128)])
        else:
            pltpu.sync_copy(zeros_hbm,
                            acc_s.at[pl.ds(tile_base, rows_per_tile)])
        plsc.subcore_barrier()

        @pl.loop(0, n_chunks // 2)
        def _(h):
            ci1 = 2 * h + 1
            ci2 = 2 * h + 2
            drain(rows0_v, gsem0)

            if async_scatter:
                @pl.when(h > 0)
                def _():
                    drain_s(idx1_v, rows1_v, ssem1)

            fire(base_g + ci1 * 2 * krow, idx1_v, rows1_v, gsem1)
            scatter(idx0_v, rows0_v, ssem0)
            drain(rows1_v, gsem1)

            @pl.when(ci2 < n_chunks)
            def _():
                if async_scatter:
                    drain_s(idx0_v, rows0_v, ssem0)
                fire(base_g + ci2 * 2 * krow, idx0_v, rows0_v, gsem0)

            scatter(idx1_v, rows1_v, ssem1)

        if async_scatter:
            drain_s(idx0_v, rows0_v, ssem0)
            drain_s(idx1_v, rows1_v, ssem1)
        plsc.subcore_barrier()
        pltpu.sync_copy(
            acc_s.at[pl.ds(tile_base, rows_per_tile)],
            out_hbm.at[cid].at[pl.ds(tile_base, rows_per_tile)])

    return segsum


@functools.lru_cache(maxsize=None)
def _make_pairsum(n_pad):
    out_per_w = n_pad // NW
    oc = 128
    n_chunks = out_per_w // oc
    mesh = plsc.VectorSubcoreMesh(core_axis_name="c", subcore_axis_name="s")

    @functools.partial(
        pl.kernel,
        out_type=jax.ShapeDtypeStruct((n_pad, D), jnp.float32),
        mesh=mesh,
        scratch_types=[
            pltpu.VMEM((2, 128), jnp.int32),
            pltpu.VMEM((2, 128), jnp.int32),
            pltpu.VMEM((2 * oc, D), jnp.float32),
            pltpu.VMEM((2 * oc, D), jnp.float32),
            pltpu.VMEM((oc, D), jnp.float32),
            pltpu.VMEM((oc, D), jnp.float32),
            pltpu.SemaphoreType.DMA,
            pltpu.SemaphoreType.DMA,
            pltpu.SemaphoreType.DMA,
            pltpu.SemaphoreType.DMA,
        ],
        compiler_params=pltpu.CompilerParams(use_tc_tiling_on_sc=False),
    )
    def pairsum(y_hbm, src_hbm, out_hbm,
                idx0_v, idx1_v, rows0_v, rows1_v, out0_v, out1_v,
                gsem0, gsem1, osem0, osem1):
        cid = lax.axis_index("c")
        sid = lax.axis_index("s")
        wid = cid * NSUB + sid
        base_out = wid * out_per_w
        base_idx = wid * n_chunks * 2

        def fire(ci, idx_v, rows_v, sem):
            pltpu.sync_copy(src_hbm.at[pl.ds(base_idx + ci * 2, 2)], idx_v)
            for j in range(2):
                pltpu.async_copy(y_hbm.at[idx_v.at[j]],
                                 rows_v.at[pl.ds(j * 128, 128)], sem)

        def drain_g(rows_v, sem):
            pltpu.make_async_copy(y_hbm.at[pl.ds(0, 2 * oc)], rows_v, sem).wait()

        def reduce_write(ci, rows_v, out_v, sem):
            @pl.loop(0, oc)
            def _(i):
                for j in range(D // 16):
                    s = pl.ds(j * 16, 16)
                    out_v[i, s] = rows_v[2 * i, s] + rows_v[2 * i + 1, s]
            pltpu.async_copy(out_v, out_hbm.at[pl.ds(base_out + ci * oc, oc)],
                             sem)

        def drain_o(out_v, sem):
            pltpu.make_async_copy(y_hbm.at[pl.ds(0, oc)], out_v, sem).wait()

        fire(0, idx0_v, rows0_v, gsem0)

        @pl.loop(0, (n_chunks + 1) // 2)
        def _(h):
            ci0 = 2 * h
            ci1 = 2 * h + 1
            ci2 = 2 * h + 2
            drain_g(rows0_v, gsem0)

            @pl.when(ci1 < n_chunks)
            def _():
                fire(ci1, idx1_v, rows1_v, gsem1)

            @pl.when(h > 0)
            def _():
                drain_o(out0_v, osem0)

            reduce_write(ci0, rows0_v, out0_v, osem0)

            @pl.when(ci1 < n_chunks)
            def _():
                drain_g(rows1_v, gsem1)

                @pl.when(ci2 < n_chunks)
                def _():
                    fire(ci2, idx0_v, rows0_v, gsem0)

                @pl.when(h > 0)
                def _():
                    drain_o(out1_v, osem1)

                reduce_write(ci1, rows1_v, out1_v, osem1)

        drain_o(out0_v, osem0)
        if n_chunks > 1:
            drain_o(out1_v, osem1)

    return pairsum


_BS = 1024


def _mm(x, w, b):
    n, din = x.shape
    dout = w.shape[1]

    def body(x_ref, w_ref, b_ref, o_ref):
        o_ref[...] = jnp.dot(x_ref[...], w_ref[...],
                             preferred_element_type=jnp.float32) + b_ref[...]

    return pl.pallas_call(
        body,
        grid=(n // _BS,),
        in_specs=[
            pl.BlockSpec((_BS, din), lambda i: (i, 0)),
            pl.BlockSpec((din, dout), lambda i: (0, 0)),
            pl.BlockSpec((1, dout), lambda i: (0, 0)),
        ],
        out_specs=pl.BlockSpec((_BS, dout), lambda i: (i, 0)),
        out_shape=jax.ShapeDtypeStruct((n, dout), jnp.float32),
    )(x, w, b.reshape(1, -1))


def _agg_spec(split):
    if split:
        nh = NHALF // _BS
        return pl.BlockSpec((1, _BS, D), lambda i: (i // nh, i % nh, 0))
    return pl.BlockSpec((NCORES, _BS, D), lambda i: (0, i, 0))


def _comb_mm(aggp, r, w, b, split=False):
    n = r.shape[0]
    dout = w.shape[1]

    def body(a_ref, r_ref, w_ref, b_ref, o_ref):
        agg = a_ref[0] if split else a_ref[0] + a_ref[1]
        h = jnp.maximum(agg + r_ref[...], 0.0)
        o_ref[...] = jnp.dot(h, w_ref[...],
                             preferred_element_type=jnp.float32) + b_ref[...]

    return pl.pallas_call(
        body,
        grid=(n // _BS,),
        in_specs=[
            _agg_spec(split),
            pl.BlockSpec((_BS, D), lambda i: (i, 0)),
            pl.BlockSpec((D, dout), lambda i: (0, 0)),
            pl.BlockSpec((1, dout), lambda i: (0, 0)),
        ],
        out_specs=pl.BlockSpec((_BS, dout), lambda i: (i, 0)),
        out_shape=jax.ShapeDtypeStruct((n, dout), jnp.float32),
    )(aggp, r, w, b.reshape(1, -1))


def _comb_pool(aggp, r, seg3d, split=False):
    n = r.shape[0]

    def body(a_ref, r_ref, s_ref, h_ref, sum_ref, cnt_ref):
        i = pl.program_id(0)
        agg = a_ref[0] if split else a_ref[0] + a_ref[1]
        h = jnp.maximum(agg + r_ref[...], 0.0)
        h_ref[...] = h
        ids = s_ref[0, 0, :]
        mask = (lax.broadcasted_iota(jnp.int32, (G, _BS), 0)
                == ids[None, :]).astype(jnp.float32)

        @pl.when(i == 0)
        def _():
            sum_ref[...] = jnp.zeros_like(sum_ref)
            cnt_ref[...] = jnp.zeros_like(cnt_ref)

        sum_ref[...] += jnp.dot(mask, h,
                                preferred_element_type=jnp.float32)
        cnt_ref[...] += jnp.sum(mask, axis=1, keepdims=True)

    return pl.pallas_call(
        body,
        grid=(n // _BS,),
        in_specs=[
            _agg_spec(split),
            pl.BlockSpec((_BS, D), lambda i: (i, 0)),
            pl.BlockSpec((1, 1, _BS), lambda i: (i, 0, 0)),
        ],
        out_specs=[
            pl.BlockSpec((_BS, D), lambda i: (i, 0)),
            pl.BlockSpec((G, D), lambda i: (0, 0)),
            pl.BlockSpec((G, 1), lambda i: (0, 0)),
        ],
        out_shape=[
            jax.ShapeDtypeStruct((n, D), jnp.float32),
            jax.ShapeDtypeStruct((G, D), jnp.float32),
            jax.ShapeDtypeStruct((G, 1), jnp.float32),
        ],
    )(aggp, r, seg3d)


def _tuple_mm(tp, iso, wa, wb, b):
    n = iso.shape[0]
    dout = wa.shape[1]

    def body(t_ref, i_ref, wa_ref, wb_ref, b_ref, o_ref):
        h = t_ref[...] * 0.5
        o_ref[...] = (jnp.dot(h, wa_ref[...], preferred_element_type=jnp.float32)
                      + jnp.dot(i_ref[...], wb_ref[...],
                                preferred_element_type=jnp.float32)
                      + b_ref[...])

    return pl.pallas_call(
        body,
        grid=(n // _BS,),
        in_specs=[
            pl.BlockSpec((_BS, D), lambda i: (i, 0)),
            pl.BlockSpec((_BS, NI2), lambda i: (i, 0)),
            pl.BlockSpec((D, dout), lambda i: (0, 0)),
            pl.BlockSpec((NI2, dout), lambda i: (0, 0)),
            pl.BlockSpec((1, dout), lambda i: (0, 0)),
        ],
        out_specs=pl.BlockSpec((_BS, dout), lambda i: (i, 0)),
        out_shape=jax.ShapeDtypeStruct((n, dout), jnp.float32),
    )(tp, iso, wa, wb, b.reshape(1, -1))


def _final_mlp(s0, c0, s1, c1, wfc0, bfc0, wfc1, bfc1, wfc2, bfc2):
    def body(s0_ref, c0_ref, s1_ref, c1_ref, w0_ref, b0_ref, w1_ref, b1_ref,
             w2_ref, b2_ref, o_ref):
        x0 = s0_ref[...] / jnp.maximum(c0_ref[...], 1.0)
        x1 = s1_ref[...] / jnp.maximum(c1_ref[...], 1.0)
        a = jnp.maximum(
            jnp.dot(x0, w0_ref[:D], preferred_element_type=jnp.float32)
            + jnp.dot(x1, w0_ref[D:], preferred_element_type=jnp.float32)
            + b0_ref[...], 0.0)
        h = jnp.maximum(
            jnp.dot(a, w1_ref[...], preferred_element_type=jnp.float32)
            + b1_ref[...], 0.0)
        o = jnp.dot(h, w2_ref[...], preferred_element_type=jnp.float32) + b2_ref[...]
        m = jnp.max(o, axis=1, keepdims=True)
        z = o - m
        lse = jnp.log(jnp.sum(jnp.exp(z), axis=1, keepdims=True))
        o_ref[...] = z - lse

    return pl.pallas_call(
        body,
        out_shape=jax.ShapeDtypeStruct((G, NC), jnp.float32),
    )(s0, c0, s1, c1, wfc0, bfc0.reshape(1, -1), wfc1, bfc1.reshape(1, -1),
      wfc2, bfc2.reshape(1, -1))


def _pad_rows(x, n_pad, fill=0.0):
    return jnp.pad(x, ((0, n_pad - x.shape[0]), (0, 0)), constant_values=fill)


def _prep_edges(src, dst, e_pad, dummy_dst, chunk):
    e = src.shape[0]
    krow = chunk // 128
    src = jnp.pad(src.astype(jnp.int32), (0, e_pad - e),
                  constant_values=0).reshape(e_pad // chunk, krow, 128)
    dst = jnp.pad(dst.astype(jnp.int32), (0, e_pad - e),
                  constant_values=dummy_dst).reshape(e_pad // chunk, krow, 128)
    return jnp.concatenate([src, dst], axis=1).reshape(-1, 128)


def kernel(x, edge_index, edge_index_2, batch, batch_2, assignment_index_2,
           iso_type_2, W_init_rel, b_init, W_init_root,
           W00_rel, b00, W00_root, W01_rel, b01, W01_root,
           W10_rel, b10, W10_root, W11_rel, b11, W11_root,
           Wfc0, bfc0, Wfc1, bfc1, Wfc2, bfc2):
    zeros1 = jnp.zeros((NPAD // NSUB, D), jnp.float32)
    zeros2 = jnp.zeros((N2PAD // NSUB, D), jnp.float32)

    eidx1 = _prep_edges(edge_index[0], edge_index[1], EPAD, N, 512)
    srca = jnp.pad(assignment_index_2[0].astype(jnp.int32),
                   (0, 2 * N2PAD - A), constant_values=0).reshape(-1, 128)
    eidx2 = _prep_edges(edge_index_2[0], edge_index_2[1], E2PAD, N2, 256)

    batch3d = jnp.pad(batch.astype(jnp.int32), (0, NPAD - N),
                      constant_values=G).reshape(NPAD // _BS, 1, _BS)
    batch23d = jnp.pad(batch_2.astype(jnp.int32), (0, N2PAD - N2),
                       constant_values=G).reshape(N2PAD // _BS, 1, _BS)

    xp = _pad_rows(x, NPAD)
    isop = _pad_rows(iso_type_2, N2PAD)

    seg1 = _make_segsum(NPAD, EPAD, 512, True, False)
    pairsum = _make_pairsum(N2PAD)
    seg2 = _make_segsum(N2PAD, E2PAD, 256, True, False)

    w_init = jnp.concatenate([W_init_rel, W_init_root], axis=1)
    b_init_c = jnp.concatenate([jnp.zeros((D,), jnp.float32), b_init])
    w00 = jnp.concatenate([W00_rel, W00_root], axis=1)
    b00_c = jnp.concatenate([jnp.zeros((D,), jnp.float32), b00])
    w01 = jnp.concatenate([W01_rel, W01_root], axis=1)
    b01_c = jnp.concatenate([jnp.zeros((D,), jnp.float32), b01])
    w10a = jnp.concatenate([W10_rel[:D], W10_root[:D]], axis=1)
    w10b = jnp.concatenate([W10_rel[D:], W10_root[D:]], axis=1)
    b10_c = jnp.concatenate([jnp.zeros((D,), jnp.float32), b10])
    w11 = jnp.concatenate([W11_rel, W11_root], axis=1)
    b11_c = jnp.concatenate([jnp.zeros((D,), jnp.float32), b11])

    yr0 = _mm(xp, w_init, b_init_c)
    y0, r0 = yr0[:, :D], yr0[:, D:]
    agg0 = seg1(y0, eidx1, zeros1)

    yr1 = _comb_mm(agg0, r0, w00, b00_c)
    y1, r1 = yr1[:, :D], yr1[:, D:]
    agg1 = seg1(y1, eidx1, zeros1)

    yr2 = _comb_mm(agg1, r1, w01, b01_c)
    y2, r2 = yr2[:, :D], yr2[:, D:]
    agg2 = seg1(y2, eidx1, zeros1)

    h3, s0, c0 = _comb_pool(agg2, r2, batch3d)

    tp = pairsum(h3, srca)
    yr4 = _tuple_mm(tp, isop, w10a, w10b, b10_c)
    y4, r4 = yr4[:, :D], yr4[:, D:]
    agg3 = seg2(y4, eidx2, zeros2)

    yr5 = _comb_mm(agg3, r4, w11, b11_c)
    y5, r5 = yr5[:, :D], yr5[:, D:]
    agg4 = seg2(y5, eidx2, zeros2)

    _, s1, c1 = _comb_pool(agg4, r5, batch23d)

    return _final_mlp(s0, c0, s1, c1, Wfc0, bfc0, Wfc1, bfc1, Wfc2, bfc2)
